# baseline (speedup 1.0000x reference)
import functools

import jax
import jax.numpy as jnp
from jax import lax
from jax.experimental import pallas as pl
from jax.experimental.pallas import tpu as pltpu
from jax.experimental.pallas import tpu_sc as plsc

N = 10000
C = 128
E = 320000
NPAD = 10016
NC = 2
NS = 16
NW = NC * NS
CHUNK = 128
CPT = 80
SEG = 16
NSEG = CPT // SEG
NCHUNKS = NW * CPT
EPAD = NCHUNKS * CHUNK
GROUPS = NPAD // 16
BASE_GR = GROUPS // NS
EXTRA_GR = GROUPS % NS


@functools.cache
def _sc_gather_scatter_kernel():
    return functools.partial(
        pl.kernel,
        mesh=plsc.VectorSubcoreMesh(core_axis_name="c", subcore_axis_name="s"),
        out_type=[
            jax.ShapeDtypeStruct((NC, NPAD, C), jnp.float32),
            jax.ShapeDtypeStruct((NC, NPAD), jnp.float32),
        ],
        scratch_types=[
            pltpu.VMEM_SHARED((NPAD, C), jnp.float32),
            pltpu.VMEM_SHARED((NPAD,), jnp.float32),
            pltpu.VMEM((2, SEG, CHUNK), jnp.int32),
            pltpu.VMEM((2, SEG, CHUNK), jnp.int32),
            pltpu.VMEM((CHUNK, C), jnp.float32),
            pltpu.VMEM((CHUNK, C), jnp.float32),
            pltpu.VMEM((CHUNK,), jnp.int32),
            pltpu.VMEM((CHUNK,), jnp.int32),
            pltpu.VMEM((CHUNK,), jnp.int32),
            pltpu.VMEM((CHUNK,), jnp.int32),
            pltpu.VMEM((16, C), jnp.float32),
            pltpu.VMEM((1024,), jnp.float32),
            pltpu.VMEM((CHUNK,), jnp.float32),
            pltpu.SemaphoreType.DMA,
            pltpu.SemaphoreType.DMA,
            pltpu.SemaphoreType.DMA,
        ],
    )(_sc_body)


def _sc_body(x_hbm, row_hbm, col_hbm, p_hbm, deg_hbm,
             acc, degs, row_t, col_t, rows_a, rows_b,
             row_va, col_va, row_vb, col_vb, zrow, zflat, ones_v,
             sg_a, sg_b, si):
    cid = lax.axis_index("c")
    sid = lax.axis_index("s")
    wid = sid * NC + cid
    start = pl.multiple_of(wid * CPT, CPT)

    def _idx_seg_start(s, slot):
        base = pl.multiple_of(start + s * SEG, SEG)
        pltpu.async_copy(row_hbm.at[pl.ds(base, SEG)], row_t.at[slot], si)
        pltpu.async_copy(col_hbm.at[pl.ds(base, SEG)], col_t.at[slot], si)

    def _idx_seg_drain(s, slot):
        base = pl.multiple_of(start + s * SEG, SEG)
        pltpu.make_async_copy(row_hbm.at[pl.ds(base, SEG)],
                              row_t.at[slot], si).wait()
        pltpu.make_async_copy(col_hbm.at[pl.ds(base, SEG)],
                              col_t.at[slot], si).wait()

    _idx_seg_start(0, 0)

    z16 = jnp.zeros((16,), jnp.float32)
    for i in range(16):
        for j in range(C // 16):
            zrow[i, pl.ds(j * 16, 16)] = z16
    for j in range(CHUNK // 16):
        ones_v[pl.ds(j * 16, 16)] = jnp.ones((16,), jnp.float32)

    def _zflat_body(i, carry):
        zflat[pl.ds(pl.multiple_of(i * 16, 16), 16)] = z16
        return carry
    lax.fori_loop(0, 1024 // 16, _zflat_body, 0)

    ngr = BASE_GR + jnp.where(sid < EXTRA_GR, 1, 0)

    def _zero_body(k, carry):
        g = sid + k * NS
        off = pl.multiple_of(g * 16, 16)
        pltpu.sync_copy(zrow, acc.at[pl.ds(off, 16)])
        return carry
    lax.fori_loop(0, ngr, _zero_body, 0)

    @pl.when(sid == 0)
    def _():
        for k in range(9):
            pltpu.sync_copy(zflat, degs.at[pl.ds(k * 1024, 1024)])
        pltpu.sync_copy(zflat.at[pl.ds(0, NPAD - 9216)],
                        degs.at[pl.ds(9216, NPAD - 9216)])

    plsc.subcore_barrier()

    def _stage_idx(slot, k, row_v, col_v):
        for j in range(CHUNK // 16):
            r = row_t[slot, k, pl.ds(j * 16, 16)]
            c = col_t[slot, k, pl.ds(j * 16, 16)]
            row_v[pl.ds(j * 16, 16)] = r
            col_v[pl.ds(j * 16, 16)] = jnp.where(r == c, N, c)

    def _gather_start(row_v, buf, sem):
        pltpu.async_copy(x_hbm.at[row_v], buf, sem)

    def _gather_drain(row_v, buf, sem):
        pltpu.make_async_copy(x_hbm.at[row_v], buf, sem).wait()

    for s in range(NSEG):
        slot = s % 2
        _idx_seg_drain(s, slot)
        if s + 1 < NSEG:
            _idx_seg_start(s + 1, 1 - slot)

        def _pipe_body(k, carry):
            _stage_idx(slot, k, row_va, col_va)
            _gather_start(row_va, rows_a, sg_a)
            _gather_drain(row_va, rows_a, sg_a)
            pltpu.sync_copy(rows_a, acc.at[col_va], add=True)
            pltpu.sync_copy(ones_v, degs.at[col_va], add=True)
            return carry
        lax.fori_loop(0, SEG, _pipe_body, 0)

    plsc.subcore_barrier()

    def _wb_body(k, carry):
        g = sid + k * NS
        off = pl.multiple_of(g * 16, 16)
        pltpu.sync_copy(acc.at[pl.ds(off, 16)], p_hbm.at[cid, pl.ds(off, 16)])
        return carry
    lax.fori_loop(0, ngr, _wb_body, 0)

    @pl.when(sid == 0)
    def _():
        pltpu.sync_copy(degs, deg_hbm.at[cid])


BN = 400


def _tc_body(x_ref, p_ref, d_ref, woutT, bout, wrootT, wfcT, bfc,
             w1T, b1, w2T, b2, woT, bo, o_ref):
    xb = x_ref[...]
    psum = p_ref[0] + p_ref[1]
    d = d_ref[...]
    deg = 1.0 + d[:, 0:1] + d[:, 1:2]
    agg = (xb + psum) / deg
    h = (jnp.dot(agg, woutT[...], preferred_element_type=jnp.float32)
         + jnp.dot(xb, wrootT[...], preferred_element_type=jnp.float32)
         + bout[...])
    h = jnp.maximum(h, 0.0)
    h = jnp.dot(h, wfcT[...], preferred_element_type=jnp.float32) + bfc[...]
    h = jnp.where(h >= 0, h, 0.01 * h)
    h = jnp.dot(h, w1T[...], preferred_element_type=jnp.float32) + b1[...]
    h = jnp.dot(h, w2T[...], preferred_element_type=jnp.float32) + b2[...]
    h = jnp.dot(h, woT[...], preferred_element_type=jnp.float32) + bo[...]
    o_ref[...] = jax.nn.sigmoid(h)


def _tc_head(x, p, dT, woutT, bout, wrootT, wfcT, bfc, w1T, b1, w2T, b2,
             woT, bo, *, interpret=False):
    grid = N // BN
    full = lambda i: (0, 0)
    return pl.pallas_call(
        _tc_body,
        grid=(grid,),
        in_specs=[
            pl.BlockSpec((BN, C), lambda i: (i, 0)),
            pl.BlockSpec((NC, BN, C), lambda i: (0, i, 0)),
            pl.BlockSpec((BN, NC), lambda i: (i, 0)),
            pl.BlockSpec((C, C), full),
            pl.BlockSpec((1, C), full),
            pl.BlockSpec((C, C), full),
            pl.BlockSpec((C, C), full),
            pl.BlockSpec((1, C), full),
            pl.BlockSpec((C, C), full),
            pl.BlockSpec((1, C), full),
            pl.BlockSpec((C, 64), full),
            pl.BlockSpec((1, 64), full),
            pl.BlockSpec((64, 6), full),
            pl.BlockSpec((1, 6), full),
        ],
        out_specs=pl.BlockSpec((BN, 6), lambda i: (i, 0)),
        out_shape=jax.ShapeDtypeStruct((N, 6), jnp.float32),
        interpret=interpret,
    )(x, p, dT, woutT, bout, wrootT, wfcT, bfc, w1T, b1, w2T, b2, woT, bo)


def kernel(x, edge_index, batch_graph, W_out, b_out, W_root, W_fc, b_fc,
           W1, b1, W2, b2, Wo, bo):
    ei = jnp.concatenate(
        [edge_index, jnp.zeros((2, EPAD - E), edge_index.dtype)], axis=1)
    row2d = ei[0].reshape(NCHUNKS, CHUNK)
    col2d = ei[1].reshape(NCHUNKS, CHUNK)
    p, dpart = _sc_gather_scatter_kernel()(x, row2d, col2d)
    return _tc_head(
        x, p, dpart.T,
        W_out.T, b_out.reshape(1, -1),
        W_root.T,
        W_fc.T, b_fc.reshape(1, -1),
        W1.T, b1.reshape(1, -1),
        W2.T, b2.reshape(1, -1),
        Wo.T, bo.reshape(1, -1),
    )


# speedup vs baseline: 1.2130x; 1.2130x over previous
import functools

import jax
import jax.numpy as jnp
from jax import lax
from jax.experimental import pallas as pl
from jax.experimental.pallas import tpu as pltpu
from jax.experimental.pallas import tpu_sc as plsc

N = 10000
C = 128
E = 320000
NPAD = 10016
NC = 2
NS = 16
NW = NC * NS
CHUNK = 128
CPT = 80
NCHUNKS = NW * CPT
EPAD = NCHUNKS * CHUNK
GROUPS = NPAD // 16
BASE_GR = GROUPS // NS
EXTRA_GR = GROUPS % NS


@functools.cache
def _sc_gather_scatter_kernel():
    return functools.partial(
        pl.kernel,
        mesh=plsc.VectorSubcoreMesh(core_axis_name="c", subcore_axis_name="s"),
        out_type=[
            jax.ShapeDtypeStruct((NC, NPAD, C), jnp.float32),
            jax.ShapeDtypeStruct((NC, NPAD), jnp.float32),
        ],
        scratch_types=[
            pltpu.VMEM_SHARED((NPAD, C), jnp.float32),
            pltpu.VMEM_SHARED((NPAD,), jnp.float32),
            pltpu.VMEM((CHUNK, C), jnp.float32),
            pltpu.VMEM((CHUNK, C), jnp.float32),
            pltpu.VMEM((CHUNK,), jnp.int32),
            pltpu.VMEM((CHUNK,), jnp.int32),
            pltpu.VMEM((CHUNK,), jnp.int32),
            pltpu.VMEM((CHUNK,), jnp.int32),
            pltpu.VMEM((16, C), jnp.float32),
            pltpu.VMEM((1024,), jnp.float32),
            pltpu.VMEM((CHUNK,), jnp.float32),
            pltpu.SemaphoreType.DMA,
            pltpu.SemaphoreType.DMA,
            pltpu.SemaphoreType.DMA,
            pltpu.SemaphoreType.DMA,
        ],
    )(_sc_body)


def _sc_body(x_hbm, row_hbm, col_hbm, p_hbm, deg_hbm,
             acc, degs, rows_a, rows_b,
             row_va, col_va, row_vb, col_vb, zrow, zflat, ones_v,
             sg_a, sg_b, si_a, si_b):
    cid = lax.axis_index("c")
    sid = lax.axis_index("s")
    wid = sid * NC + cid

    def _coff(k):
        return pl.multiple_of((wid + k * NW) * CHUNK, CHUNK)

    def _idx_start(k, row_v, col_v, sem):
        off = _coff(k)
        pltpu.async_copy(row_hbm.at[pl.ds(off, CHUNK)], row_v, sem)
        pltpu.async_copy(col_hbm.at[pl.ds(off, CHUNK)], col_v, sem)

    def _idx_drain(k, row_v, col_v, sem):
        off = _coff(k)
        pltpu.make_async_copy(row_hbm.at[pl.ds(off, CHUNK)], row_v, sem).wait()
        pltpu.make_async_copy(col_hbm.at[pl.ds(off, CHUNK)], col_v, sem).wait()

    _idx_start(0, row_va, col_va, si_a)
    _idx_start(1, row_vb, col_vb, si_b)

    z16 = jnp.zeros((16,), jnp.float32)
    for i in range(16):
        for j in range(C // 16):
            zrow[i, pl.ds(j * 16, 16)] = z16
    for j in range(CHUNK // 16):
        ones_v[pl.ds(j * 16, 16)] = jnp.ones((16,), jnp.float32)

    def _zflat_body(i, carry):
        zflat[pl.ds(pl.multiple_of(i * 16, 16), 16)] = z16
        return carry
    lax.fori_loop(0, 1024 // 16, _zflat_body, 0)

    ngr = BASE_GR + jnp.where(sid < EXTRA_GR, 1, 0)

    def _zero_body(k, carry):
        g = sid + k * NS
        off = pl.multiple_of(g * 16, 16)
        pltpu.sync_copy(zrow, acc.at[pl.ds(off, 16)])
        return carry
    lax.fori_loop(0, ngr, _zero_body, 0)

    @pl.when(sid == 0)
    def _():
        for k in range(9):
            pltpu.sync_copy(zflat, degs.at[pl.ds(k * 1024, 1024)])
        pltpu.sync_copy(zflat.at[pl.ds(0, NPAD - 9216)],
                        degs.at[pl.ds(9216, NPAD - 9216)])

    plsc.subcore_barrier()

    def _gather_start(row_v, buf, sem):
        pltpu.async_copy(x_hbm.at[row_v], buf, sem)

    def _gather_drain(row_v, buf, sem):
        pltpu.make_async_copy(x_hbm.at[row_v], buf, sem).wait()

    def _mask(row_v, col_v):
        for j in range(CHUNK // 16):
            r = row_v[pl.ds(j * 16, 16)]
            c = col_v[pl.ds(j * 16, 16)]
            col_v[pl.ds(j * 16, 16)] = jnp.where(r == c, N, c)

    _idx_drain(0, row_va, col_va, si_a)
    _gather_start(row_va, rows_a, sg_a)
    _mask(row_va, col_va)
    _idx_drain(1, row_vb, col_vb, si_b)
    _gather_start(row_vb, rows_b, sg_b)
    _mask(row_vb, col_vb)

    def _pipe_body(t, carry):
        k0 = t * 2
        k1 = k0 + 1
        _gather_drain(row_va, rows_a, sg_a)
        pltpu.sync_copy(rows_a, acc.at[col_va], add=True)
        pltpu.sync_copy(ones_v, degs.at[col_va], add=True)

        @pl.when(t < CPT // 2 - 1)
        def _():
            _idx_start(k0 + 2, row_va, col_va, si_a)
        _gather_drain(row_vb, rows_b, sg_b)
        pltpu.sync_copy(rows_b, acc.at[col_vb], add=True)
        pltpu.sync_copy(ones_v, degs.at[col_vb], add=True)

        @pl.when(t < CPT // 2 - 1)
        def _():
            _idx_start(k0 + 3, row_vb, col_vb, si_b)
            _idx_drain(k0 + 2, row_va, col_va, si_a)
            _gather_start(row_va, rows_a, sg_a)
            _mask(row_va, col_va)
            _idx_drain(k0 + 3, row_vb, col_vb, si_b)
            _gather_start(row_vb, rows_b, sg_b)
            _mask(row_vb, col_vb)
        return carry
    lax.fori_loop(0, CPT // 2, _pipe_body, 0)

    plsc.subcore_barrier()

    def _wb_body(k, carry):
        g = sid + k * NS
        off = pl.multiple_of(g * 16, 16)
        pltpu.sync_copy(acc.at[pl.ds(off, 16)], p_hbm.at[cid, pl.ds(off, 16)])
        return carry
    lax.fori_loop(0, ngr, _wb_body, 0)

    @pl.when(sid == 0)
    def _():
        pltpu.sync_copy(degs, deg_hbm.at[cid])


BN = 400


def _tc_body(x_ref, p_ref, d_ref, woutT, bout, wrootT, wfcT, bfc,
             w1T, b1, w2T, b2, woT, bo, o_ref):
    xb = x_ref[...]
    psum = p_ref[0] + p_ref[1]
    d = d_ref[...]
    deg = 1.0 + d[:, 0:1] + d[:, 1:2]
    agg = (xb + psum) / deg
    h = (jnp.dot(agg, woutT[...], preferred_element_type=jnp.float32)
         + jnp.dot(xb, wrootT[...], preferred_element_type=jnp.float32)
         + bout[...])
    h = jnp.maximum(h, 0.0)
    h = jnp.dot(h, wfcT[...], preferred_element_type=jnp.float32) + bfc[...]
    h = jnp.where(h >= 0, h, 0.01 * h)
    h = jnp.dot(h, w1T[...], preferred_element_type=jnp.float32) + b1[...]
    h = jnp.dot(h, w2T[...], preferred_element_type=jnp.float32) + b2[...]
    h = jnp.dot(h, woT[...], preferred_element_type=jnp.float32) + bo[...]
    o_ref[...] = jax.nn.sigmoid(h)


def _tc_head(x, p, dT, woutT, bout, wrootT, wfcT, bfc, w1T, b1, w2T, b2,
             woT, bo, *, interpret=False):
    grid = N // BN
    full = lambda i: (0, 0)
    return pl.pallas_call(
        _tc_body,
        grid=(grid,),
        in_specs=[
            pl.BlockSpec((BN, C), lambda i: (i, 0)),
            pl.BlockSpec((NC, BN, C), lambda i: (0, i, 0)),
            pl.BlockSpec((BN, NC), lambda i: (i, 0)),
            pl.BlockSpec((C, C), full),
            pl.BlockSpec((1, C), full),
            pl.BlockSpec((C, C), full),
            pl.BlockSpec((C, C), full),
            pl.BlockSpec((1, C), full),
            pl.BlockSpec((C, C), full),
            pl.BlockSpec((1, C), full),
            pl.BlockSpec((C, 64), full),
            pl.BlockSpec((1, 64), full),
            pl.BlockSpec((64, 6), full),
            pl.BlockSpec((1, 6), full),
        ],
        out_specs=pl.BlockSpec((BN, 6), lambda i: (i, 0)),
        out_shape=jax.ShapeDtypeStruct((N, 6), jnp.float32),
        interpret=interpret,
    )(x, p, dT, woutT, bout, wrootT, wfcT, bfc, w1T, b1, w2T, b2, woT, bo)


def kernel(x, edge_index, batch_graph, W_out, b_out, W_root, W_fc, b_fc,
           W1, b1, W2, b2, Wo, bo):
    ei = jnp.concatenate(
        [edge_index, jnp.zeros((2, EPAD - E), edge_index.dtype)], axis=1)
    p, dpart = _sc_gather_scatter_kernel()(x, ei[0], ei[1])
    return _tc_head(
        x, p, dpart.T,
        W_out.T, b_out.reshape(1, -1),
        W_root.T,
        W_fc.T, b_fc.reshape(1, -1),
        W1.T, b1.reshape(1, -1),
        W2.T, b2.reshape(1, -1),
        Wo.T, bo.reshape(1, -1),
    )


# speedup vs baseline: 2.6916x; 2.2189x over previous
import functools

import jax
import jax.numpy as jnp
from jax import lax
from jax.experimental import pallas as pl
from jax.experimental.pallas import tpu as pltpu
from jax.experimental.pallas import tpu_sc as plsc

N = 10000
C = 128
E = 320000
NPAD = 10016
NC = 2
NS = 16
NW = NC * NS
CHUNK = 128
NCHUNKS = E // CHUNK
BASE_CH = NCHUNKS // NW
EXTRA_CH = NCHUNKS % NW
GROUPS = NPAD // 16
BASE_GR = GROUPS // NS
EXTRA_GR = GROUPS % NS


@functools.cache
def _sc_gather_scatter_kernel():
    return functools.partial(
        pl.kernel,
        mesh=plsc.VectorSubcoreMesh(core_axis_name="c", subcore_axis_name="s"),
        out_type=[
            jax.ShapeDtypeStruct((NC, NPAD, C), jnp.float32),
            jax.ShapeDtypeStruct((NC, NPAD), jnp.float32),
        ],
        scratch_types=[
            pltpu.VMEM_SHARED((NPAD, C), jnp.float32),
            pltpu.VMEM_SHARED((NPAD,), jnp.float32),
            pltpu.VMEM((CHUNK, C), jnp.float32),
            pltpu.VMEM((CHUNK, C), jnp.float32),
            pltpu.VMEM((CHUNK,), jnp.int32),
            pltpu.VMEM((CHUNK,), jnp.int32),
            pltpu.VMEM((CHUNK,), jnp.int32),
            pltpu.VMEM((CHUNK,), jnp.int32),
            pltpu.VMEM((16, C), jnp.float32),
            pltpu.VMEM((1024,), jnp.float32),
            pltpu.VMEM((CHUNK,), jnp.float32),
            pltpu.SemaphoreType.DMA,
            pltpu.SemaphoreType.DMA,
            pltpu.SemaphoreType.DMA,
            pltpu.SemaphoreType.DMA,
        ],
    )(_sc_body)


def _sc_body(x_hbm, row_hbm, col_hbm, p_hbm, deg_hbm,
             acc, degs, rows_a, rows_b,
             row_va, col_va, row_vb, col_vb, zrow, zflat, ones_v,
             sg_a, sg_b, si_a, si_b):
    cid = lax.axis_index("c")
    sid = lax.axis_index("s")
    wid = sid * NC + cid

    def _coff(k):
        return pl.multiple_of((wid + k * NW) * CHUNK, CHUNK)

    def _idx_start(k, row_v, col_v, sem):
        off = _coff(k)
        pltpu.async_copy(row_hbm.at[pl.ds(off, CHUNK)], row_v, sem)
        pltpu.async_copy(col_hbm.at[pl.ds(off, CHUNK)], col_v, sem)

    def _idx_drain(k, row_v, col_v, sem):
        off = _coff(k)
        pltpu.make_async_copy(row_hbm.at[pl.ds(off, CHUNK)], row_v, sem).wait()
        pltpu.make_async_copy(col_hbm.at[pl.ds(off, CHUNK)], col_v, sem).wait()

    _idx_start(0, row_va, col_va, si_a)
    _idx_start(1, row_vb, col_vb, si_b)

    z16 = jnp.zeros((16,), jnp.float32)
    for i in range(16):
        for j in range(C // 16):
            zrow[i, pl.ds(j * 16, 16)] = z16
    for j in range(CHUNK // 16):
        ones_v[pl.ds(j * 16, 16)] = jnp.ones((16,), jnp.float32)

    def _zflat_body(i, carry):
        zflat[pl.ds(pl.multiple_of(i * 16, 16), 16)] = z16
        return carry
    lax.fori_loop(0, 1024 // 16, _zflat_body, 0)

    ngr = BASE_GR + jnp.where(sid < EXTRA_GR, 1, 0)

    def _zero_body(k, carry):
        g = sid + k * NS
        off = pl.multiple_of(g * 16, 16)
        pltpu.sync_copy(zrow, acc.at[pl.ds(off, 16)])
        return carry
    lax.fori_loop(0, ngr, _zero_body, 0)

    @pl.when(sid == 0)
    def _():
        for k in range(9):
            pltpu.sync_copy(zflat, degs.at[pl.ds(k * 1024, 1024)])
        pltpu.sync_copy(zflat.at[pl.ds(0, NPAD - 9216)],
                        degs.at[pl.ds(9216, NPAD - 9216)])

    plsc.subcore_barrier()

    def _gather_start(row_v, buf, sem):
        pltpu.async_copy(x_hbm.at[row_v], buf, sem)

    def _gather_drain(row_v, buf, sem):
        pltpu.make_async_copy(x_hbm.at[row_v], buf, sem).wait()

    def _mask(row_v, col_v):
        for j in range(CHUNK // 16):
            r = row_v[pl.ds(j * 16, 16)]
            c = col_v[pl.ds(j * 16, 16)]
            col_v[pl.ds(j * 16, 16)] = jnp.where(r == c, N, c)

    nch = BASE_CH + jnp.where(wid < EXTRA_CH, 1, 0)

    _idx_drain(0, row_va, col_va, si_a)
    _gather_start(row_va, rows_a, sg_a)
    _mask(row_va, col_va)
    _idx_drain(1, row_vb, col_vb, si_b)
    _gather_start(row_vb, rows_b, sg_b)
    _mask(row_vb, col_vb)

    def _pipe_body(t, carry):
        k0 = t * 2
        k1 = k0 + 1
        _gather_drain(row_va, rows_a, sg_a)
        pltpu.sync_copy(rows_a, acc.at[col_va], add=True)
        pltpu.sync_copy(ones_v, degs.at[col_va], add=True)

        @pl.when(k0 + 2 < nch)
        def _():
            _idx_start(k0 + 2, row_va, col_va, si_a)
        _gather_drain(row_vb, rows_b, sg_b)
        pltpu.sync_copy(rows_b, acc.at[col_vb], add=True)
        pltpu.sync_copy(ones_v, degs.at[col_vb], add=True)

        @pl.when(k1 + 2 < nch)
        def _():
            _idx_start(k1 + 2, row_vb, col_vb, si_b)

        @pl.when(k0 + 2 < nch)
        def _():
            _idx_drain(k0 + 2, row_va, col_va, si_a)
            _gather_start(row_va, rows_a, sg_a)
            _mask(row_va, col_va)

        @pl.when(k1 + 2 < nch)
        def _():
            _idx_drain(k1 + 2, row_vb, col_vb, si_b)
            _gather_start(row_vb, rows_b, sg_b)
            _mask(row_vb, col_vb)
        return carry
    lax.fori_loop(0, BASE_CH // 2, _pipe_body, 0)

    @pl.when(wid < EXTRA_CH)
    def _():
        _gather_drain(row_va, rows_a, sg_a)
        pltpu.sync_copy(rows_a, acc.at[col_va], add=True)
        pltpu.sync_copy(ones_v, degs.at[col_va], add=True)

    plsc.subcore_barrier()

    def _wb_body(k, carry):
        g = sid + k * NS
        off = pl.multiple_of(g * 16, 16)
        pltpu.sync_copy(acc.at[pl.ds(off, 16)], p_hbm.at[cid, pl.ds(off, 16)])
        return carry
    lax.fori_loop(0, ngr, _wb_body, 0)

    @pl.when(sid == 0)
    def _():
        pltpu.sync_copy(degs, deg_hbm.at[cid])


BN = 400


def _tc_body(x_ref, p_ref, d_ref, woutT, bout, wrootT, wfcT, bfc,
             w1T, b1, w2T, b2, woT, bo, o_ref):
    xb = x_ref[...]
    psum = p_ref[0] + p_ref[1]
    d = d_ref[...]
    deg = 1.0 + d[:, 0:1] + d[:, 1:2]
    agg = (xb + psum) / deg
    h = (jnp.dot(agg, woutT[...], preferred_element_type=jnp.float32)
         + jnp.dot(xb, wrootT[...], preferred_element_type=jnp.float32)
         + bout[...])
    h = jnp.maximum(h, 0.0)
    h = jnp.dot(h, wfcT[...], preferred_element_type=jnp.float32) + bfc[...]
    h = jnp.where(h >= 0, h, 0.01 * h)
    h = jnp.dot(h, w1T[...], preferred_element_type=jnp.float32) + b1[...]
    h = jnp.dot(h, w2T[...], preferred_element_type=jnp.float32) + b2[...]
    h = jnp.dot(h, woT[...], preferred_element_type=jnp.float32) + bo[...]
    o_ref[...] = jax.nn.sigmoid(h)


def _tc_head(x, p, dT, woutT, bout, wrootT, wfcT, bfc, w1T, b1, w2T, b2,
             woT, bo, *, interpret=False):
    grid = N // BN
    full = lambda i: (0, 0)
    return pl.pallas_call(
        _tc_body,
        grid=(grid,),
        in_specs=[
            pl.BlockSpec((BN, C), lambda i: (i, 0)),
            pl.BlockSpec((NC, BN, C), lambda i: (0, i, 0)),
            pl.BlockSpec((BN, NC), lambda i: (i, 0)),
            pl.BlockSpec((C, C), full),
            pl.BlockSpec((1, C), full),
            pl.BlockSpec((C, C), full),
            pl.BlockSpec((C, C), full),
            pl.BlockSpec((1, C), full),
            pl.BlockSpec((C, C), full),
            pl.BlockSpec((1, C), full),
            pl.BlockSpec((C, 64), full),
            pl.BlockSpec((1, 64), full),
            pl.BlockSpec((64, 6), full),
            pl.BlockSpec((1, 6), full),
        ],
        out_specs=pl.BlockSpec((BN, 6), lambda i: (i, 0)),
        out_shape=jax.ShapeDtypeStruct((N, 6), jnp.float32),
        interpret=interpret,
    )(x, p, dT, woutT, bout, wrootT, wfcT, bfc, w1T, b1, w2T, b2, woT, bo)


def kernel(x, edge_index, batch_graph, W_out, b_out, W_root, W_fc, b_fc,
           W1, b1, W2, b2, Wo, bo):
    p, dpart = _sc_gather_scatter_kernel()(x, edge_index[0], edge_index[1])
    return _tc_head(
        x, p, dpart.T,
        W_out.T, b_out.reshape(1, -1),
        W_root.T,
        W_fc.T, b_fc.reshape(1, -1),
        W1.T, b1.reshape(1, -1),
        W2.T, b2.reshape(1, -1),
        Wo.T, bo.reshape(1, -1),
    )
